# baseline (speedup 1.0000x reference)
import dataclasses
import functools

import numpy as np

import jax
import jax.numpy as jnp
from jax.experimental import pallas as pl
from jax.experimental.pallas import tpu as pltpu
from jax.experimental.pallas import tpu_sc as plsc

NCORES = 2
NTILES = 16
LANES = 16


def _make_sc_edge_kernel(npad, epad, DF, H, CP, G, WB, UNROLL):
    AW = DF + LANES
    ept = epad // (NCORES * NTILES)
    nchunk = ept // G
    rows_per_tile = npad // NTILES
    assert ept % G == 0
    mesh = plsc.VectorSubcoreMesh(core_axis_name="c", subcore_axis_name="s")
    cp = pltpu.CompilerParams()
    if "needs_layout_passes" in pltpu.CompilerParams.__dataclass_fields__:
        cp = dataclasses.replace(cp, needs_layout_passes=False)
    if "use_tc_tiling_on_sc" in pltpu.CompilerParams.__dataclass_fields__:
        cp = dataclasses.replace(cp, use_tc_tiling_on_sc=False)

    R = max(r for r in (128, 80, 64, 40, 32, 16) if rows_per_tile % r == 0
            and r <= G)
    assert nchunk % 4 == 0

    @functools.partial(
        pl.kernel,
        mesh=mesh,
        compiler_params=cp,
        out_type=jax.ShapeDtypeStruct((NCORES, npad, AW), jnp.float32),
        scratch_types=[
            pltpu.VMEM((4, G), jnp.int32),
            pltpu.VMEM((4, G), jnp.int32),
            pltpu.VMEM((2, G, DF), jnp.float32),
            pltpu.VMEM((2, G, DF), jnp.float32),
            pltpu.VMEM((WB, G, AW), jnp.float32),
            pltpu.VMEM((DF,), jnp.float32),
            pltpu.VMEM_SHARED((npad, AW), jnp.float32),
            pltpu.SemaphoreType.DMA,
            pltpu.SemaphoreType.DMA,
            pltpu.SemaphoreType.DMA,
            pltpu.SemaphoreType.DMA,
            pltpu.SemaphoreType.DMA,
            pltpu.SemaphoreType.DMA,
            pltpu.SemaphoreType.DMA,
            pltpu.SemaphoreType.DMA,
        ],
    )
    def sc_kernel(xl_hbm, xr_hbm, src_hbm, dst_hbm, att_hbm, out_hbm,
                  srcv, dstv, glv, grv, wv, attv, acc,
                  sem_i0, sem_i1, sem_i2, sem_i3, sem_g0, sem_g1,
                  sem_s0, sem_s1):
        cid = jax.lax.axis_index("c")
        sid = jax.lax.axis_index("s")
        wid = cid * NTILES + sid
        sem_i = (sem_i0, sem_i1, sem_i2, sem_i3)
        sem_g = (sem_g0, sem_g1)
        sem_s = (sem_s0, sem_s1)

        zero16 = jnp.zeros((LANES,), jnp.float32)

        @pl.loop(0, G)
        def _(i):
            for w in range(WB):
                for j in range(AW // LANES):
                    wv[w, i, pl.ds(LANES * j, LANES)] = zero16

        row0 = sid * rows_per_tile

        @pl.loop(0, rows_per_tile // R)
        def _(j):
            pltpu.sync_copy(wv.at[0, pl.ds(0, R)],
                            acc.at[pl.ds(row0 + j * R, R)])

        pltpu.sync_copy(att_hbm, attv)
        att_chunks = [attv[pl.ds(LANES * j, LANES)] for j in range(DF // LANES)]
        lane_iota = jax.lax.iota(jnp.int32, LANES)

        plsc.subcore_barrier()

        ebase = wid * ept

        def start_idx(g, b4):
            base = ebase + g * G
            pltpu.async_copy(src_hbm.at[pl.ds(base, G)], srcv.at[b4],
                             sem_i[b4])
            pltpu.async_copy(dst_hbm.at[pl.ds(base, G)], dstv.at[b4],
                             sem_i[b4])

        def wait_idx(g, b4):
            base = ebase + g * G
            pltpu.make_async_copy(src_hbm.at[pl.ds(base, G)], srcv.at[b4],
                                  sem_i[b4]).wait()
            pltpu.make_async_copy(dst_hbm.at[pl.ds(base, G)], dstv.at[b4],
                                  sem_i[b4]).wait()

        def start_gathers(b2, b4):
            pltpu.async_copy(xl_hbm.at[srcv.at[b4]], glv.at[b2], sem_g[b2])
            pltpu.async_copy(xr_hbm.at[dstv.at[b4]], grv.at[b2], sem_g[b2])

        def wait_gathers(b2, b4):
            pltpu.make_async_copy(xl_hbm.at[srcv.at[b4]], glv.at[b2],
                                  sem_g[b2]).wait()
            pltpu.make_async_copy(xr_hbm.at[dstv.at[b4]], grv.at[b2],
                                  sem_g[b2]).wait()

        def start_scatter(wb, b4):
            pltpu.async_copy(wv.at[wb], acc.at[dstv.at[b4]], sem_s[wb],
                             add=True)

        def wait_scatter(wb, b4):
            pltpu.make_async_copy(wv.at[wb], acc.at[dstv.at[b4]],
                                  sem_s[wb]).wait()

        def shuf(v, idx):
            return v.at[idx].get(mode="promise_in_bounds")

        X8 = lane_iota ^ 8
        X4 = lane_iota ^ 4
        X2 = lane_iota ^ 2
        X1 = lane_iota ^ 1
        lt8 = lane_iota < 8
        and4 = (lane_iota & 4) == 0
        and2 = (lane_iota & 2) == 0
        inv = (0, 8, 4, 12, 2, 10, 6, 14)

        def lrelu_att(e):
            ts = []
            for k in range(DF // LANES):
                a = glv[b2_cl[0], e, pl.ds(LANES * k, LANES)]
                bb = grv[b2_cl[0], e, pl.ds(LANES * k, LANES)]
                u = a + bb
                v = jnp.maximum(u, u * jnp.float32(0.2))
                ts.append(v * att_chunks[k])
            return ts

        b2_cl = [0]

        def edge_h8(e, wb):
            ts = lrelu_att(e)
            aa = [t + shuf(t, X8) for t in ts]
            bb = [jnp.where(lt8, aa[2 * j], shuf(aa[2 * j + 1], X8))
                  for j in range(4)]
            cc = [b + shuf(b, X4) for b in bb]
            dd = [jnp.where(and4, cc[2 * k], shuf(cc[2 * k + 1], X4))
                  for k in range(2)]
            ee = [d + shuf(d, X2) for d in dd]
            ff = jnp.where(and2, ee[0], shuf(ee[1], X2))
            gsum = ff + shuf(ff, X1)
            pall = jnp.exp(gsum)
            wv[wb, e, pl.ds(DF, LANES)] = pall
            for h in range(8):
                pv = shuf(pall, jnp.full((LANES,), inv[h], jnp.int32))
                a = glv[b2_cl[0], e, pl.ds(LANES * h, LANES)]
                wv[wb, e, pl.ds(LANES * h, LANES)] = pv * a

        def edge_h1(e, wb):
            ts = lrelu_att(e)
            s = ts[0]
            for t in ts[1:]:
                s = s + t
            for X in (X8, X4, X2, X1):
                s = s + shuf(s, X)
            pall = jnp.exp(s)
            wv[wb, e, pl.ds(DF, LANES)] = pall
            for k in range(DF // LANES):
                a = glv[b2_cl[0], e, pl.ds(LANES * k, LANES)]
                wv[wb, e, pl.ds(LANES * k, LANES)] = pall * a

        edge_body = edge_h8 if H == 8 else edge_h1
        assert H in (1, 8)

        def compute(b2, wb):
            b2_cl[0] = b2

            @plsc.parallel_loop(0, G, unroll=UNROLL)
            def _(e):
                edge_body(e, wb)

        pltpu.sync_copy(src_hbm.at[pl.ds(ebase, G)], srcv.at[0])
        pltpu.sync_copy(dst_hbm.at[pl.ds(ebase, G)], dstv.at[0])
        start_gathers(0, 0)
        start_idx(1, 1)

        @pl.loop(0, nchunk // 4)
        def _(gp):
            for j in range(4):
                g = 4 * gp + j
                b2, b4, wb = j % 2, j, j % WB
                @pl.when(g + 1 < nchunk)
                def _():
                    wait_idx(g + 1, (j + 1) % 4)
                    start_gathers((j + 1) % 2, (j + 1) % 4)
                wait_gathers(b2, b4)
                @pl.when(g >= WB)
                def _():
                    wait_scatter((j - WB) % WB, (j - WB) % 4)
                compute(b2, wb)
                start_scatter(wb, b4)
                @pl.when(g + 2 < nchunk)
                def _():
                    start_idx(g + 2, (j + 2) % 4)

        for t in range(WB):
            g = nchunk - WB + t
            wait_scatter(g % WB, g % 4)

        plsc.subcore_barrier()

        @pl.loop(0, rows_per_tile // R)
        def _(j):
            r = row0 + j * R
            pltpu.sync_copy(acc.at[pl.ds(r, R)], wv.at[0, pl.ds(0, R)])
            pltpu.sync_copy(wv.at[0, pl.ds(0, R)],
                            out_hbm.at[cid, pl.ds(r, R)])

    return sc_kernel


def _mm2(x, wl, wr, bm):
    npad, d = x.shape
    dout = wl.shape[1]

    def body(x_ref, wl_ref, wr_ref, xl_ref, xr_ref):
        xv = x_ref[...]
        xl_ref[...] = jnp.dot(xv, wl_ref[...],
                              preferred_element_type=jnp.float32)
        xr_ref[...] = jnp.dot(xv, wr_ref[...],
                              preferred_element_type=jnp.float32)

    return pl.pallas_call(
        body,
        grid=(npad // bm,),
        in_specs=[
            pl.BlockSpec((bm, d), lambda i: (i, 0)),
            pl.BlockSpec((d, dout), lambda i: (0, 0)),
            pl.BlockSpec((d, dout), lambda i: (0, 0)),
        ],
        out_specs=[
            pl.BlockSpec((bm, dout), lambda i: (i, 0)),
            pl.BlockSpec((bm, dout), lambda i: (i, 0)),
        ],
        out_shape=[
            jax.ShapeDtypeStruct((npad, dout), jnp.float32),
            jax.ShapeDtypeStruct((npad, dout), jnp.float32),
        ],
    )(x, wl, wr)


def _mid(accu, zsel, b1, w2l, w2r, bm):
    _, npad, AW = accu.shape
    DF = AW - LANES
    dout = w2l.shape[1]

    def body(a_ref, zsel_ref, b1_ref, wl_ref, wr_ref, yl_ref, yr_ref):
        u = a_ref[0] + a_ref[1]
        zs = u[:, DF:DF + LANES]
        zexp = jnp.dot(zs, zsel_ref[...], preferred_element_type=jnp.float32)
        h = u[:, :DF] / (zexp + jnp.float32(1e-16)) + b1_ref[...]
        h = jnp.where(h > 0, h, jnp.exp(jnp.minimum(h, 0.0)) - jnp.float32(1.0))
        yl_ref[...] = jnp.dot(h, wl_ref[...],
                              preferred_element_type=jnp.float32)
        yr_ref[...] = jnp.dot(h, wr_ref[...],
                              preferred_element_type=jnp.float32)

    return pl.pallas_call(
        body,
        grid=(npad // bm,),
        in_specs=[
            pl.BlockSpec((2, bm, AW), lambda i: (0, i, 0)),
            pl.BlockSpec((LANES, DF), lambda i: (0, 0)),
            pl.BlockSpec((1, DF), lambda i: (0, 0)),
            pl.BlockSpec((DF, dout), lambda i: (0, 0)),
            pl.BlockSpec((DF, dout), lambda i: (0, 0)),
        ],
        out_specs=[
            pl.BlockSpec((bm, dout), lambda i: (i, 0)),
            pl.BlockSpec((bm, dout), lambda i: (i, 0)),
        ],
        out_shape=[
            jax.ShapeDtypeStruct((npad, dout), jnp.float32),
            jax.ShapeDtypeStruct((npad, dout), jnp.float32),
        ],
    )(accu, zsel, b1, w2l, w2r)


def _final(accu, b2, bm, DF, nvalid):
    _, npad, AW = accu.shape

    def body(a_ref, b2_ref, o_ref):
        u = a_ref[0] + a_ref[1]
        z = u[:, DF:DF + 1]
        o = u / (z + jnp.float32(1e-16)) + b2_ref[...]
        ci = jax.lax.broadcasted_iota(jnp.int32, (bm, AW), 1)
        valid = ci < nvalid
        neg = jnp.float32(-1e30)
        m = jnp.max(jnp.where(valid, o, neg), axis=1, keepdims=True)
        ex = jnp.where(valid, jnp.exp(o - m), jnp.float32(0.0))
        lse = jnp.log(jnp.sum(ex, axis=1, keepdims=True))
        o_ref[...] = (o - m) - lse

    return pl.pallas_call(
        body,
        grid=(npad // bm,),
        in_specs=[
            pl.BlockSpec((2, bm, AW), lambda i: (0, i, 0)),
            pl.BlockSpec((1, AW), lambda i: (0, 0)),
        ],
        out_specs=pl.BlockSpec((bm, AW), lambda i: (i, 0)),
        out_shape=jax.ShapeDtypeStruct((npad, AW), jnp.float32),
    )(accu, b2)


def kernel(x, edge_index, W1l, W1r, att1, b1, W2l, W2r, att2, b2):
    N, D = x.shape
    E = edge_index.shape[1]
    H1, C1 = att1.shape
    DOUT = W2l.shape[1]

    NPAD = 10240
    EP = E + N
    G1, G2 = 48, 96
    EPW = NCORES * NTILES * G1 * 2
    EPAD = ((EP + EPW - 1) // EPW) * EPW
    DF2 = ((DOUT + LANES - 1) // LANES) * LANES
    AW1 = H1 * C1 + LANES
    AW2 = DF2 + LANES
    BM = 2048

    loop = jnp.arange(N, dtype=edge_index.dtype)
    pad_e = jnp.full((EPAD - EP,), N, dtype=edge_index.dtype)
    src = jnp.concatenate([edge_index[0], loop, pad_e])
    dst = jnp.concatenate([edge_index[1], loop, pad_e])

    xpad = jnp.zeros((NPAD, D), jnp.float32).at[:N].set(x)
    att1f = att1.reshape(H1 * C1)
    att2f = jnp.zeros((DF2,), jnp.float32).at[:DOUT].set(att2.reshape(DOUT))
    w2l_p = jnp.zeros((H1 * C1, DF2), jnp.float32).at[:, :DOUT].set(W2l)
    w2r_p = jnp.zeros((H1 * C1, DF2), jnp.float32).at[:, :DOUT].set(W2r)
    b1_p = b1.reshape(1, H1 * C1)
    b2_p = jnp.zeros((1, AW2), jnp.float32).at[0, :DOUT].set(b2)

    headmap = (0, 0, 4, 4, 2, 2, 6, 6, 1, 1, 5, 5, 3, 3, 7, 7)
    zsel = np.zeros((LANES, H1 * C1), np.float32)
    for L in range(0, LANES, 2):
        zsel[L, headmap[L] * C1:(headmap[L] + 1) * C1] = 1.0
    zsel = jnp.asarray(zsel)

    xl, xr = _mm2(xpad, W1l, W1r, BM)
    sc1 = _make_sc_edge_kernel(NPAD, EPAD, H1 * C1, H1, C1, G1, WB=1,
                               UNROLL=4)
    accu1 = sc1(xl, xr, src, dst, att1f)

    yl, yr = _mid(accu1, zsel, b1_p, w2l_p, w2r_p, BM)
    sc2 = _make_sc_edge_kernel(NPAD, EPAD, DF2, 1, DF2, G2, WB=2, UNROLL=8)
    accu2 = sc2(yl, yr, src, dst, att2f)

    out = _final(accu2, b2_p, BM, DF2, DOUT)
    return out[:N, :DOUT]


# speedup vs baseline: 1.3133x; 1.3133x over previous
import dataclasses
import functools

import numpy as np

import jax
import jax.numpy as jnp
from jax.experimental import pallas as pl
from jax.experimental.pallas import tpu as pltpu
from jax.experimental.pallas import tpu_sc as plsc

NCORES = 2
NTILES = 16
LANES = 16


def _make_sc_edge_kernel(npad, epad, DF, H, CP, G, WB, UNROLL):
    AW = DF + LANES
    ept = epad // (NCORES * NTILES)
    nchunk = ept // G
    rows_per_tile = npad // NTILES
    assert ept % G == 0
    mesh = plsc.VectorSubcoreMesh(core_axis_name="c", subcore_axis_name="s")
    cp = pltpu.CompilerParams()
    if "needs_layout_passes" in pltpu.CompilerParams.__dataclass_fields__:
        cp = dataclasses.replace(cp, needs_layout_passes=False)
    if "use_tc_tiling_on_sc" in pltpu.CompilerParams.__dataclass_fields__:
        cp = dataclasses.replace(cp, use_tc_tiling_on_sc=False)

    R = max(r for r in (128, 80, 64, 40, 32, 16) if rows_per_tile % r == 0
            and r <= G)
    assert nchunk % 4 == 0

    @functools.partial(
        pl.kernel,
        mesh=mesh,
        compiler_params=cp,
        out_type=jax.ShapeDtypeStruct((NCORES, npad, AW), jnp.float32),
        scratch_types=[
            pltpu.VMEM((4, G), jnp.int32),
            pltpu.VMEM((4, G), jnp.int32),
            pltpu.VMEM((2, G, DF), jnp.float32),
            pltpu.VMEM((2, G, DF), jnp.float32),
            pltpu.VMEM((WB, G, AW), jnp.float32),
            pltpu.VMEM((DF,), jnp.float32),
            pltpu.VMEM_SHARED((npad, AW), jnp.float32),
            pltpu.SemaphoreType.DMA,
            pltpu.SemaphoreType.DMA,
            pltpu.SemaphoreType.DMA,
            pltpu.SemaphoreType.DMA,
            pltpu.SemaphoreType.DMA,
            pltpu.SemaphoreType.DMA,
            pltpu.SemaphoreType.DMA,
            pltpu.SemaphoreType.DMA,
        ],
    )
    def sc_kernel(xl_hbm, xr_hbm, src_hbm, dst_hbm, att_hbm, out_hbm,
                  srcv, dstv, glv, grv, wv, attv, acc,
                  sem_i0, sem_i1, sem_i2, sem_i3, sem_g0, sem_g1,
                  sem_s0, sem_s1):
        cid = jax.lax.axis_index("c")
        sid = jax.lax.axis_index("s")
        wid = cid * NTILES + sid
        sem_i = (sem_i0, sem_i1, sem_i2, sem_i3)
        sem_g = (sem_g0, sem_g1)
        sem_s = (sem_s0, sem_s1)

        zero16 = jnp.zeros((LANES,), jnp.float32)

        @pl.loop(0, G)
        def _(i):
            for w in range(WB):
                for j in range(AW // LANES):
                    wv[w, i, pl.ds(LANES * j, LANES)] = zero16

        row0 = sid * rows_per_tile

        @pl.loop(0, rows_per_tile // R)
        def _(j):
            pltpu.sync_copy(wv.at[0, pl.ds(0, R)],
                            acc.at[pl.ds(row0 + j * R, R)])

        pltpu.sync_copy(att_hbm, attv)
        att_chunks = [attv[pl.ds(LANES * j, LANES)] for j in range(DF // LANES)]
        lane_iota = jax.lax.iota(jnp.int32, LANES)

        plsc.subcore_barrier()

        ebase = wid * ept

        def start_idx(g, b4):
            base = ebase + g * G
            pltpu.async_copy(src_hbm.at[pl.ds(base, G)], srcv.at[b4],
                             sem_i[b4])
            pltpu.async_copy(dst_hbm.at[pl.ds(base, G)], dstv.at[b4],
                             sem_i[b4])

        def wait_idx(g, b4):
            base = ebase + g * G
            pltpu.make_async_copy(src_hbm.at[pl.ds(base, G)], srcv.at[b4],
                                  sem_i[b4]).wait()
            pltpu.make_async_copy(dst_hbm.at[pl.ds(base, G)], dstv.at[b4],
                                  sem_i[b4]).wait()

        def start_gathers(b2, b4):
            pltpu.async_copy(xl_hbm.at[srcv.at[b4]], glv.at[b2], sem_g[b2])
            pltpu.async_copy(xr_hbm.at[dstv.at[b4]], grv.at[b2], sem_g[b2])

        def wait_gathers(b2, b4):
            pltpu.make_async_copy(xl_hbm.at[srcv.at[b4]], glv.at[b2],
                                  sem_g[b2]).wait()
            pltpu.make_async_copy(xr_hbm.at[dstv.at[b4]], grv.at[b2],
                                  sem_g[b2]).wait()

        def start_scatter(wb, b4):
            pltpu.async_copy(wv.at[wb], acc.at[dstv.at[b4]], sem_s[wb],
                             add=True)

        def wait_scatter(wb, b4):
            pltpu.make_async_copy(wv.at[wb], acc.at[dstv.at[b4]],
                                  sem_s[wb]).wait()

        def shuf(v, idx):
            return v.at[idx].get(mode="promise_in_bounds")

        X8 = lane_iota ^ 8
        X4 = lane_iota ^ 4
        X2 = lane_iota ^ 2
        X1 = lane_iota ^ 1
        lt8 = lane_iota < 8
        and4 = (lane_iota & 4) == 0
        and2 = (lane_iota & 2) == 0
        inv = (0, 8, 4, 12, 2, 10, 6, 14)

        def lrelu_att(e):
            ts = []
            for k in range(DF // LANES):
                a = glv[b2_cl[0], e, pl.ds(LANES * k, LANES)]
                bb = grv[b2_cl[0], e, pl.ds(LANES * k, LANES)]
                u = a + bb
                v = jnp.maximum(u, u * jnp.float32(0.2))
                ts.append(v * att_chunks[k])
            return ts

        b2_cl = [0]

        def edge_h8_scan(e, wb):
            zvec = jnp.zeros((LANES,), jnp.float32)
            for h in range(8):
                a = glv[b2_cl[0], e, pl.ds(LANES * h, LANES)]
                bb = grv[b2_cl[0], e, pl.ds(LANES * h, LANES)]
                u = a + bb
                v = jnp.maximum(u, u * jnp.float32(0.2))
                s = jnp.sum(v * att_chunks[h])
                pv = jnp.exp(jnp.full((LANES,), s, jnp.float32))
                wv[wb, e, pl.ds(LANES * h, LANES)] = pv * a
                zvec = zvec + jnp.where(lane_iota == h, pv, jnp.float32(0.0))
            wv[wb, e, pl.ds(DF, LANES)] = zvec

        def edge_h8(e, wb):
            ts = lrelu_att(e)
            aa = [t + shuf(t, X8) for t in ts]
            bb = [jnp.where(lt8, aa[2 * j], shuf(aa[2 * j + 1], X8))
                  for j in range(4)]
            cc = [b + shuf(b, X4) for b in bb]
            dd = [jnp.where(and4, cc[2 * k], shuf(cc[2 * k + 1], X4))
                  for k in range(2)]
            ee = [d + shuf(d, X2) for d in dd]
            ff = jnp.where(and2, ee[0], shuf(ee[1], X2))
            gsum = ff + shuf(ff, X1)
            pall = jnp.exp(gsum)
            wv[wb, e, pl.ds(DF, LANES)] = pall
            for h in range(8):
                pv = shuf(pall, jnp.full((LANES,), inv[h], jnp.int32))
                a = glv[b2_cl[0], e, pl.ds(LANES * h, LANES)]
                wv[wb, e, pl.ds(LANES * h, LANES)] = pv * a

        def edge_h1(e, wb):
            ts = lrelu_att(e)
            s = ts[0]
            for t in ts[1:]:
                s = s + t
            for X in (X8, X4, X2, X1):
                s = s + shuf(s, X)
            pall = jnp.exp(s)
            wv[wb, e, pl.ds(DF, LANES)] = pall
            for k in range(DF // LANES):
                a = glv[b2_cl[0], e, pl.ds(LANES * k, LANES)]
                wv[wb, e, pl.ds(LANES * k, LANES)] = pall * a

        edge_body = edge_h8_scan if H == 8 else edge_h1
        assert H in (1, 8)

        def compute(b2, wb):
            b2_cl[0] = b2

            @plsc.parallel_loop(0, G, unroll=UNROLL)
            def _(e):
                edge_body(e, wb)

        pltpu.sync_copy(src_hbm.at[pl.ds(ebase, G)], srcv.at[0])
        pltpu.sync_copy(dst_hbm.at[pl.ds(ebase, G)], dstv.at[0])
        start_gathers(0, 0)
        start_idx(1, 1)

        @pl.loop(0, nchunk // 4)
        def _(gp):
            for j in range(4):
                g = 4 * gp + j
                b2, b4, wb = j % 2, j, j % WB
                @pl.when(g + 1 < nchunk)
                def _():
                    wait_idx(g + 1, (j + 1) % 4)
                    start_gathers((j + 1) % 2, (j + 1) % 4)
                wait_gathers(b2, b4)
                @pl.when(g >= WB)
                def _():
                    wait_scatter((j - WB) % WB, (j - WB) % 4)
                compute(b2, wb)
                start_scatter(wb, b4)
                @pl.when(g + 2 < nchunk)
                def _():
                    start_idx(g + 2, (j + 2) % 4)

        for t in range(WB):
            g = nchunk - WB + t
            wait_scatter(g % WB, g % 4)

        plsc.subcore_barrier()

        @pl.loop(0, rows_per_tile // R)
        def _(j):
            r = row0 + j * R
            pltpu.sync_copy(acc.at[pl.ds(r, R)], wv.at[0, pl.ds(0, R)])
            pltpu.sync_copy(wv.at[0, pl.ds(0, R)],
                            out_hbm.at[cid, pl.ds(r, R)])

    return sc_kernel


def _mm2(x, wl, wr, bm):
    npad, d = x.shape
    dout = wl.shape[1]

    def body(x_ref, wl_ref, wr_ref, xl_ref, xr_ref):
        xv = x_ref[...]
        xl_ref[...] = jnp.dot(xv, wl_ref[...],
                              preferred_element_type=jnp.float32)
        xr_ref[...] = jnp.dot(xv, wr_ref[...],
                              preferred_element_type=jnp.float32)

    return pl.pallas_call(
        body,
        grid=(npad // bm,),
        in_specs=[
            pl.BlockSpec((bm, d), lambda i: (i, 0)),
            pl.BlockSpec((d, dout), lambda i: (0, 0)),
            pl.BlockSpec((d, dout), lambda i: (0, 0)),
        ],
        out_specs=[
            pl.BlockSpec((bm, dout), lambda i: (i, 0)),
            pl.BlockSpec((bm, dout), lambda i: (i, 0)),
        ],
        out_shape=[
            jax.ShapeDtypeStruct((npad, dout), jnp.float32),
            jax.ShapeDtypeStruct((npad, dout), jnp.float32),
        ],
    )(x, wl, wr)


def _mid(accu, zsel, b1, w2l, w2r, bm):
    _, npad, AW = accu.shape
    DF = AW - LANES
    dout = w2l.shape[1]

    def body(a_ref, zsel_ref, b1_ref, wl_ref, wr_ref, yl_ref, yr_ref):
        u = a_ref[0] + a_ref[1]
        zs = u[:, DF:DF + LANES]
        zexp = jnp.dot(zs, zsel_ref[...], preferred_element_type=jnp.float32)
        h = u[:, :DF] / (zexp + jnp.float32(1e-16)) + b1_ref[...]
        h = jnp.where(h > 0, h, jnp.exp(jnp.minimum(h, 0.0)) - jnp.float32(1.0))
        yl_ref[...] = jnp.dot(h, wl_ref[...],
                              preferred_element_type=jnp.float32)
        yr_ref[...] = jnp.dot(h, wr_ref[...],
                              preferred_element_type=jnp.float32)

    return pl.pallas_call(
        body,
        grid=(npad // bm,),
        in_specs=[
            pl.BlockSpec((2, bm, AW), lambda i: (0, i, 0)),
            pl.BlockSpec((LANES, DF), lambda i: (0, 0)),
            pl.BlockSpec((1, DF), lambda i: (0, 0)),
            pl.BlockSpec((DF, dout), lambda i: (0, 0)),
            pl.BlockSpec((DF, dout), lambda i: (0, 0)),
        ],
        out_specs=[
            pl.BlockSpec((bm, dout), lambda i: (i, 0)),
            pl.BlockSpec((bm, dout), lambda i: (i, 0)),
        ],
        out_shape=[
            jax.ShapeDtypeStruct((npad, dout), jnp.float32),
            jax.ShapeDtypeStruct((npad, dout), jnp.float32),
        ],
    )(accu, zsel, b1, w2l, w2r)


def _final(accu, b2, bm, DF, nvalid):
    _, npad, AW = accu.shape

    def body(a_ref, b2_ref, o_ref):
        u = a_ref[0] + a_ref[1]
        z = u[:, DF:DF + 1]
        o = u / (z + jnp.float32(1e-16)) + b2_ref[...]
        ci = jax.lax.broadcasted_iota(jnp.int32, (bm, AW), 1)
        valid = ci < nvalid
        neg = jnp.float32(-1e30)
        m = jnp.max(jnp.where(valid, o, neg), axis=1, keepdims=True)
        ex = jnp.where(valid, jnp.exp(o - m), jnp.float32(0.0))
        lse = jnp.log(jnp.sum(ex, axis=1, keepdims=True))
        o_ref[...] = (o - m) - lse

    return pl.pallas_call(
        body,
        grid=(npad // bm,),
        in_specs=[
            pl.BlockSpec((2, bm, AW), lambda i: (0, i, 0)),
            pl.BlockSpec((1, AW), lambda i: (0, 0)),
        ],
        out_specs=pl.BlockSpec((bm, AW), lambda i: (i, 0)),
        out_shape=jax.ShapeDtypeStruct((npad, AW), jnp.float32),
    )(accu, b2)


def kernel(x, edge_index, W1l, W1r, att1, b1, W2l, W2r, att2, b2):
    N, D = x.shape
    E = edge_index.shape[1]
    H1, C1 = att1.shape
    DOUT = W2l.shape[1]

    NPAD = 10240
    EP = E + N
    G1, G2 = 48, 96
    EPW = NCORES * NTILES * G1 * 2
    EPAD = ((EP + EPW - 1) // EPW) * EPW
    DF2 = ((DOUT + LANES - 1) // LANES) * LANES
    AW1 = H1 * C1 + LANES
    AW2 = DF2 + LANES
    BM = 2048

    loop = jnp.arange(N, dtype=edge_index.dtype)
    pad_e = jnp.full((EPAD - EP,), N, dtype=edge_index.dtype)
    src = jnp.concatenate([edge_index[0], loop, pad_e])
    dst = jnp.concatenate([edge_index[1], loop, pad_e])

    xpad = jnp.zeros((NPAD, D), jnp.float32).at[:N].set(x)
    att1f = att1.reshape(H1 * C1)
    att2f = jnp.zeros((DF2,), jnp.float32).at[:DOUT].set(att2.reshape(DOUT))
    w2l_p = jnp.zeros((H1 * C1, DF2), jnp.float32).at[:, :DOUT].set(W2l)
    w2r_p = jnp.zeros((H1 * C1, DF2), jnp.float32).at[:, :DOUT].set(W2r)
    b1_p = b1.reshape(1, H1 * C1)
    b2_p = jnp.zeros((1, AW2), jnp.float32).at[0, :DOUT].set(b2)

    zsel = np.zeros((LANES, H1 * C1), np.float32)
    for h in range(H1):
        zsel[h, h * C1:(h + 1) * C1] = 1.0
    zsel = jnp.asarray(zsel)

    xl, xr = _mm2(xpad, W1l, W1r, BM)
    sc1 = _make_sc_edge_kernel(NPAD, EPAD, H1 * C1, H1, C1, G1, WB=1,
                               UNROLL=6)
    accu1 = sc1(xl, xr, src, dst, att1f)

    yl, yr = _mid(accu1, zsel, b1_p, w2l_p, w2r_p, BM)
    sc2 = _make_sc_edge_kernel(NPAD, EPAD, DF2, 1, DF2, G2, WB=2, UNROLL=8)
    accu2 = sc2(yl, yr, src, dst, att2f)

    out = _final(accu2, b2_p, BM, DF2, DOUT)
    return out[:N, :DOUT]
